# Initial kernel scaffold; baseline (speedup 1.0000x reference)
import jax
import jax.numpy as jnp
from jax.experimental import pallas as pl


def kernel(x, latent_vector1, latent_vector2, edge_index, edge_attr, batch_size, nroi, fc1_W, fc1_b, fc1_g, fc1_be, fc2_W, fc2_b, fc2_g, fc2_be, fc3_W, fc3_b, fc3_g, fc3_be, fc4_W, fc4_b, fc4_g, fc4_be, g1_W0, g1_W1, g1_b, g1_g, g1_be, g2_W0, g2_W1, g2_b, g2_g, g2_be):
    raise NotImplementedError("write your pallas kernel here")


# speedup vs baseline: 4.5129x; 4.5129x over previous
import functools

import jax
import jax.numpy as jnp
from jax import lax
from jax.experimental import pallas as pl
from jax.experimental.pallas import tpu as pltpu
from jax.experimental.pallas import tpu_sc as plsc

F32 = jnp.float32
_NS = 16
_C = 128


def _norm_act(h_raw, stats, g, be, n):
    mu = stats[0:1, :] / n
    var = stats[1:2, :] / n - mu * mu
    rs = lax.rsqrt(var + 1e-5)
    hn = (h_raw - mu) * rs * g + be
    return jnp.where(hn >= 0, hn, 0.01 * hn)


def _acc_stats(acc_ref, st_ref, h, i, nsteps):
    @pl.when(i == 0)
    def _():
        acc_ref[...] = jnp.zeros_like(acc_ref)

    acc_ref[0:1, :] = acc_ref[0:1, :] + jnp.sum(h, axis=0, keepdims=True)
    acc_ref[1:2, :] = acc_ref[1:2, :] + jnp.sum(h * h, axis=0, keepdims=True)

    @pl.when(i == nsteps - 1)
    def _():
        st_ref[...] = acc_ref[...]


def _first_layer(x, lv_rep, Wa, Wb, b, R):
    N, din = x.shape
    F = Wa.shape[1]
    NB = N // R

    def body(x_ref, lv_ref, wa_ref, wb_ref, b_ref, out_ref, st_ref, acc_ref):
        i = pl.program_id(0)
        h = (jnp.dot(x_ref[...], wa_ref[...], preferred_element_type=F32)
             + jnp.dot(lv_ref[...], wb_ref[...], preferred_element_type=F32)
             + b_ref[...])
        out_ref[...] = h
        _acc_stats(acc_ref, st_ref, h, i, NB)

    return pl.pallas_call(
        body,
        grid=(NB,),
        in_specs=[
            pl.BlockSpec((R, din), lambda i: (i, 0)),
            pl.BlockSpec((R, lv_rep.shape[1]), lambda i: (i, 0)),
            pl.BlockSpec(Wa.shape, lambda i: (0, 0)),
            pl.BlockSpec(Wb.shape, lambda i: (0, 0)),
            pl.BlockSpec((1, F), lambda i: (0, 0)),
        ],
        out_specs=[
            pl.BlockSpec((R, F), lambda i: (i, 0)),
            pl.BlockSpec((8, F), lambda i: (0, 0)),
        ],
        out_shape=[
            jax.ShapeDtypeStruct((N, F), F32),
            jax.ShapeDtypeStruct((8, F), F32),
        ],
        scratch_shapes=[pltpu.VMEM((8, F), F32)],
    )(x, lv_rep, Wa, Wb, b)


def _mid_layer(h_raw, st, g, be, W, b, R, lv_rep=None, Wb=None):
    N, Fin = h_raw.shape
    F = W.shape[1]
    NB = N // R
    has_lv = lv_rep is not None

    def body(*refs):
        if has_lv:
            (h_ref, st_ref, g_ref, be_ref, w_ref, b_ref, lv_ref, wb_ref,
             out_ref, sto_ref, acc_ref) = refs
        else:
            (h_ref, st_ref, g_ref, be_ref, w_ref, b_ref,
             out_ref, sto_ref, acc_ref) = refs
        i = pl.program_id(0)
        hn = _norm_act(h_ref[...], st_ref[...], g_ref[...], be_ref[...],
                       float(N))
        h = jnp.dot(hn, w_ref[...], preferred_element_type=F32) + b_ref[...]
        if has_lv:
            h = h + jnp.dot(lv_ref[...], wb_ref[...],
                            preferred_element_type=F32)
        out_ref[...] = h
        _acc_stats(acc_ref, sto_ref, h, i, NB)

    in_specs = [
        pl.BlockSpec((R, Fin), lambda i: (i, 0)),
        pl.BlockSpec((8, Fin), lambda i: (0, 0)),
        pl.BlockSpec((1, Fin), lambda i: (0, 0)),
        pl.BlockSpec((1, Fin), lambda i: (0, 0)),
        pl.BlockSpec((Fin, F), lambda i: (0, 0)),
        pl.BlockSpec((1, F), lambda i: (0, 0)),
    ]
    args = [h_raw, st, g, be, W, b]
    if has_lv:
        in_specs += [
            pl.BlockSpec((R, lv_rep.shape[1]), lambda i: (i, 0)),
            pl.BlockSpec(Wb.shape, lambda i: (0, 0)),
        ]
        args += [lv_rep, Wb]

    return pl.pallas_call(
        body,
        grid=(NB,),
        in_specs=in_specs,
        out_specs=[
            pl.BlockSpec((R, F), lambda i: (i, 0)),
            pl.BlockSpec((8, F), lambda i: (0, 0)),
        ],
        out_shape=[
            jax.ShapeDtypeStruct((N, F), F32),
            jax.ShapeDtypeStruct((8, F), F32),
        ],
        scratch_shapes=[pltpu.VMEM((8, F), F32)],
    )(*args)


def _conv_pre(h_raw, st, g, be, W1a, W1b, W0, b0, R):
    N, Fin = h_raw.shape
    Dh = W1a.shape[1]
    F0 = W0.shape[1]
    NB = N // R

    def body(h_ref, st_ref, g_ref, be_ref, w1a_ref, w1b_ref, w0_ref, b0_ref,
             ya_ref, yb_ref, d_ref):
        hn = _norm_act(h_ref[...], st_ref[...], g_ref[...], be_ref[...],
                       float(N))
        ya_ref[...] = jnp.dot(hn, w1a_ref[...], preferred_element_type=F32)
        yb_ref[...] = jnp.dot(hn, w1b_ref[...], preferred_element_type=F32)
        d_ref[...] = (jnp.dot(hn, w0_ref[...], preferred_element_type=F32)
                      + b0_ref[...])

    return pl.pallas_call(
        body,
        grid=(NB,),
        in_specs=[
            pl.BlockSpec((R, Fin), lambda i: (i, 0)),
            pl.BlockSpec((8, Fin), lambda i: (0, 0)),
            pl.BlockSpec((1, Fin), lambda i: (0, 0)),
            pl.BlockSpec((1, Fin), lambda i: (0, 0)),
            pl.BlockSpec((Fin, Dh), lambda i: (0, 0)),
            pl.BlockSpec((Fin, Dh), lambda i: (0, 0)),
            pl.BlockSpec((Fin, F0), lambda i: (0, 0)),
            pl.BlockSpec((1, F0), lambda i: (0, 0)),
        ],
        out_specs=[
            pl.BlockSpec((R, Dh), lambda i: (i, 0)),
            pl.BlockSpec((R, Dh), lambda i: (i, 0)),
            pl.BlockSpec((R, F0), lambda i: (i, 0)),
        ],
        out_shape=[
            jax.ShapeDtypeStruct((N, Dh), F32),
            jax.ShapeDtypeStruct((N, Dh), F32),
            jax.ShapeDtypeStruct((N, F0), F32),
        ],
    )(h_raw, st, g, be, W1a, W1b, W0, b0)


def _add_stats(d, sA, sB, R):
    N, F = d.shape
    Dh = sA.shape[1]
    NB = N // R

    def body(d_ref, sa_ref, sb_ref, out_ref, st_ref, acc_ref):
        i = pl.program_id(0)
        s = jnp.concatenate([sa_ref[...], sb_ref[...]], axis=1)
        h = d_ref[...] + s
        out_ref[...] = h
        _acc_stats(acc_ref, st_ref, h, i, NB)

    return pl.pallas_call(
        body,
        grid=(NB,),
        in_specs=[
            pl.BlockSpec((R, F), lambda i: (i, 0)),
            pl.BlockSpec((R, Dh), lambda i: (i, 0)),
            pl.BlockSpec((R, Dh), lambda i: (i, 0)),
        ],
        out_specs=[
            pl.BlockSpec((R, F), lambda i: (i, 0)),
            pl.BlockSpec((8, F), lambda i: (0, 0)),
        ],
        out_shape=[
            jax.ShapeDtypeStruct((N, F), F32),
            jax.ShapeDtypeStruct((8, F), F32),
        ],
        scratch_shapes=[pltpu.VMEM((8, F), F32)],
    )(d, sA, sB)


def _final_layer(h_raw, st, g, be, dep, R):
    N, F = h_raw.shape
    NB = N // R

    def body(h_ref, st_ref, g_ref, be_ref, dep_ref, out_ref):
        hn = _norm_act(h_ref[...], st_ref[...], g_ref[...], be_ref[...],
                       float(N))
        out_ref[...] = hn + dep_ref[0, 0]

    return pl.pallas_call(
        body,
        grid=(NB,),
        in_specs=[
            pl.BlockSpec((R, F), lambda i: (i, 0)),
            pl.BlockSpec((8, F), lambda i: (0, 0)),
            pl.BlockSpec((1, F), lambda i: (0, 0)),
            pl.BlockSpec((1, F), lambda i: (0, 0)),
            pl.BlockSpec(memory_space=pltpu.SMEM),
        ],
        out_specs=pl.BlockSpec((R, F), lambda i: (i, 0)),
        out_shape=jax.ShapeDtypeStruct((N, F), F32),
    )(h_raw, st, g, be, dep)


def _sc_mesh():
    return plsc.VectorSubcoreMesh(core_axis_name="c", subcore_axis_name="s",
                                  num_cores=2, num_subcores=_NS)


def _sc_norm(row2, col2, w2, n_nodes):
    EP2, C = row2.shape
    RPT = EP2 // _NS
    GRP = RPT * C // 16
    NDR = (n_nodes + 15) // 16
    NDRP = -(-NDR // _NS) * _NS
    SL = NDRP // _NS

    @functools.partial(
        pl.kernel,
        out_type=jax.ShapeDtypeStruct((EP2, C), F32),
        mesh=_sc_mesh(),
        compiler_params=pltpu.CompilerParams(needs_layout_passes=False, use_tc_tiling_on_sc=False),
        scratch_types=[
            pltpu.VMEM((RPT, C), jnp.int32),
            pltpu.VMEM((RPT, C), jnp.int32),
            pltpu.VMEM((RPT, C), F32),
            pltpu.VMEM((NDRP, 16), F32),
            pltpu.VMEM((RPT, C), F32),
            pltpu.VMEM((SL, 16), F32),
            pltpu.VMEM((SL, 16), F32),
            pltpu.VMEM_SHARED((_NS, NDRP, 16), F32),
            pltpu.VMEM_SHARED((NDRP, 16), F32),
        ],
    )
    def k(row_hbm, col_hbm, w_hbm, norm_hbm,
          rloc, cloc, wloc, degloc, nloc, tmp, accs, deg_all, deg_sum):
        cid = lax.axis_index("c")
        sid = lax.axis_index("s")
        base = sid * RPT
        pltpu.sync_copy(row_hbm.at[pl.ds(base, RPT)], rloc)
        pltpu.sync_copy(col_hbm.at[pl.ds(base, RPT)], cloc)
        pltpu.sync_copy(w_hbm.at[pl.ds(base, RPT)], wloc)

        zero16 = jnp.zeros((16,), F32)

        def zrow(j, _):
            degloc[j, :] = zero16
            return 0
        lax.fori_loop(0, NDRP, zrow, 0)

        def acc_edge(g, _):
            r = g >> 3
            l = (g & 7) * 16
            r16 = rloc[r, pl.ds(l, 16)]
            w16 = wloc[r, pl.ds(l, 16)]
            plsc.addupdate_scatter(degloc, [r16 >> 4, r16 & 15], w16)
            return 0
        lax.fori_loop(0, GRP, acc_edge, 0)

        pltpu.sync_copy(degloc, deg_all.at[sid])
        plsc.subcore_barrier()

        def zs(j, _):
            accs[j, :] = zero16
            return 0
        lax.fori_loop(0, SL, zs, 0)

        def red_one(kk, _):
            pltpu.sync_copy(deg_all.at[kk, pl.ds(sid * SL, SL)], tmp)

            def addrow(j, _):
                accs[j, :] = accs[j, :] + tmp[j, :]
                return 0
            lax.fori_loop(0, SL, addrow, 0)
            return 0
        lax.fori_loop(0, _NS, red_one, 0)

        pltpu.sync_copy(accs, deg_sum.at[pl.ds(sid * SL, SL)])
        plsc.subcore_barrier()
        pltpu.sync_copy(deg_sum, degloc)

        def disrow(j, _):
            v = degloc[j, :]
            vc = jnp.maximum(v, 1e-12)
            yi = jnp.int32(0x5F3759DF) - (plsc.bitcast(vc, jnp.int32) >> 1)
            y = plsc.bitcast(yi, F32)
            half = 0.5 * vc
            y = y * (1.5 - half * y * y)
            y = y * (1.5 - half * y * y)
            y = y * (1.5 - half * y * y)
            degloc[j, :] = jnp.where(v > 0, y, 0.0)
            return 0
        lax.fori_loop(0, NDRP, disrow, 0)

        def norm_edge(g, _):
            r = g >> 3
            l = (g & 7) * 16
            r16 = rloc[r, pl.ds(l, 16)]
            c16 = cloc[r, pl.ds(l, 16)]
            w16 = wloc[r, pl.ds(l, 16)]
            dr = plsc.load_gather(degloc, [r16 >> 4, r16 & 15])
            dc = plsc.load_gather(degloc, [c16 >> 4, c16 & 15])
            nloc[r, pl.ds(l, 16)] = -(dr * w16 * dc)
            return 0
        lax.fori_loop(0, GRP, norm_edge, 0)

        @pl.when(cid == 0)
        def _():
            pltpu.sync_copy(nloc, norm_hbm.at[pl.ds(base, RPT)])

    return k(row2, col2, w2)


def _sc_spmm(tabA, tabB, nrm2, row2, col2, n_nodes):
    EP2, C = row2.shape
    RPT = EP2 // _NS
    Dh = tabA.shape[1]
    RT = -(-n_nodes // (_NS * 80)) * 80
    NROW = RT * _NS
    OCH = 80
    JV = Dh // 16
    G8 = 8

    @functools.partial(
        pl.kernel,
        out_type=[
            jax.ShapeDtypeStruct((n_nodes, Dh), F32),
            jax.ShapeDtypeStruct((n_nodes, Dh), F32),
        ],
        mesh=_sc_mesh(),
        compiler_params=pltpu.CompilerParams(needs_layout_passes=False, use_tc_tiling_on_sc=False),
        scratch_types=[
            pltpu.VMEM((G8, C), jnp.int32),
            pltpu.VMEM((G8, C), jnp.int32),
            pltpu.VMEM((G8, C), F32),
            pltpu.VMEM((C, Dh), F32),
            pltpu.VMEM_SHARED((NROW, Dh), F32),
            pltpu.SemaphoreType.DMA,
        ],
    )
    def k(tA, tB, nrm_hbm, row_hbm, col_hbm, outA, outB,
          rloc, cloc, nloc, gbuf, acc, sem):
        cid = lax.axis_index("c")
        sid = lax.axis_index("s")
        base = sid * RPT

        zero16 = jnp.zeros((16,), F32)

        def zg(e, _):
            for j in range(JV):
                gbuf[e, pl.ds(16 * j, 16)] = zero16
            return 0
        lax.fori_loop(0, C, zg, 0)

        rt0 = sid * RT
        for q in range(RT // C):
            pltpu.sync_copy(gbuf, acc.at[pl.ds(rt0 + q * C, C)])
        plsc.subcore_barrier()

        def group(go, _):
            gb = base + go * G8
            pltpu.sync_copy(row_hbm.at[pl.ds(gb, G8)], rloc)
            pltpu.sync_copy(col_hbm.at[pl.ds(gb, G8)], cloc)
            pltpu.sync_copy(nrm_hbm.at[pl.ds(gb, G8)], nloc)

            def chunk(g, _):
                @pl.when(cid == 0)
                def _():
                    pltpu.async_copy(tA.at[cloc.at[g]], gbuf, sem).wait()

                @pl.when(cid == 1)
                def _():
                    pltpu.async_copy(tB.at[cloc.at[g]], gbuf, sem).wait()

                def sc_e(q, _):
                    nv = nloc[g, pl.ds(16 * q, 16)]
                    for kk in range(16):
                        e = 16 * q + kk
                        s = nv[kk]
                        for j in range(JV):
                            gbuf[e, pl.ds(16 * j, 16)] = (
                                gbuf[e, pl.ds(16 * j, 16)] * s)
                    return 0
                lax.fori_loop(0, C // 16, sc_e, 0)

                pltpu.sync_copy(gbuf, acc.at[rloc.at[g]], add=True)
                return 0
            lax.fori_loop(0, G8, chunk, 0)
            return 0
        lax.fori_loop(0, RPT // G8, group, 0)
        plsc.subcore_barrier()

        for q in range(RT // OCH):
            b2 = rt0 + q * OCH

            @pl.when(b2 + OCH <= n_nodes)
            def _():
                pltpu.sync_copy(acc.at[pl.ds(b2, OCH)], gbuf.at[pl.ds(0, OCH)])

                @pl.when(cid == 0)
                def _():
                    pltpu.sync_copy(gbuf.at[pl.ds(0, OCH)],
                                    outA.at[pl.ds(b2, OCH)])

                @pl.when(cid == 1)
                def _():
                    pltpu.sync_copy(gbuf.at[pl.ds(0, OCH)],
                                    outB.at[pl.ds(b2, OCH)])

    return k(tabA, tabB, nrm2, row2, col2)


def kernel(x, latent_vector1, latent_vector2, edge_index, edge_attr,
           batch_size, nroi,
           fc1_W, fc1_b, fc1_g, fc1_be, fc2_W, fc2_b, fc2_g, fc2_be,
           fc3_W, fc3_b, fc3_g, fc3_be, fc4_W, fc4_b, fc4_g, fc4_be,
           g1_W0, g1_W1, g1_b, g1_g, g1_be, g2_W0, g2_W1, g2_b, g2_g, g2_be):
    N, din = x.shape
    bs, dlat = latent_vector1.shape
    nr = N // bs
    f1 = fc1_W.shape[0]
    f2 = fc3_W.shape[0]
    dout = g2_W0.shape[1]
    E = edge_attr.shape[0]
    R = 1000
    assert N % R == 0 and N % 16 == 0

    row1d = lambda a: a.reshape(1, -1)

    cpt = -(-E // (_NS * _C))
    cpt = -(-cpt // 8) * 8
    per_tile = cpt * _C
    EP = per_tile * _NS
    pad = EP - E
    rowp = jnp.concatenate([edge_index[0], jnp.zeros((pad,), jnp.int32)])
    colp = jnp.concatenate([edge_index[1], jnp.zeros((pad,), jnp.int32)])
    wp = jnp.concatenate([edge_attr, jnp.zeros((pad,), F32)])
    row2 = rowp.reshape(-1, _C)
    col2 = colp.reshape(-1, _C)
    w2 = wp.reshape(-1, _C)

    nrm2 = _sc_norm(row2, col2, w2, N)

    lv1_rep = jnp.repeat(latent_vector1, nr, axis=0)
    lv2_rep = jnp.repeat(latent_vector2, nr, axis=0)

    h1, s1 = _first_layer(x, lv1_rep, fc1_W[:din], fc1_W[din:],
                          row1d(fc1_b), R)
    h2, s2 = _mid_layer(h1, s1, row1d(fc1_g), row1d(fc1_be),
                        fc2_W, row1d(fc2_b), R)
    h3, s3 = _mid_layer(h2, s2, row1d(fc2_g), row1d(fc2_be),
                        fc3_W[:f1], row1d(fc3_b), R,
                        lv_rep=lv2_rep, Wb=fc3_W[f1:])
    h4, s4 = _mid_layer(h3, s3, row1d(fc3_g), row1d(fc3_be),
                        fc4_W, row1d(fc4_b), R)

    hd1 = f2 // 2
    yA, yB, d1 = _conv_pre(h4, s4, row1d(fc4_g), row1d(fc4_be),
                           g1_W1[:, :hd1], g1_W1[:, hd1:], g1_W0,
                           row1d(g1_b), R)
    S1A, S1B = _sc_spmm(yA, yB, nrm2, row2, col2, N)
    p1, s5 = _add_stats(d1, S1A, S1B, R)

    hd2 = dout // 2
    y2A, y2B, d2 = _conv_pre(p1, s5, row1d(g1_g), row1d(g1_be),
                             g2_W1[:, :hd2], g2_W1[:, hd2:], g2_W0,
                             row1d(g2_b), R)
    S2A, S2B = _sc_spmm(y2A, y2B, nrm2, row2, col2, N)
    p2, s6 = _add_stats(d2, S2A, S2B, R)

    dep = ((jnp.asarray(batch_size) - bs) + (jnp.asarray(nroi) - nr))
    dep = dep.astype(F32).reshape(1, 1)
    out = _final_layer(p2, s6, row1d(g2_g), row1d(g2_be), dep, R)
    return out.reshape(bs, nr, dout)
